# Initial kernel scaffold; baseline (speedup 1.0000x reference)
import jax
import jax.numpy as jnp
from jax.experimental import pallas as pl


def kernel(i, j, wi, wj, bi, bj):
    raise NotImplementedError("write your pallas kernel here")


# speedup vs baseline: 1.1616x; 1.1616x over previous
import functools

import jax
import jax.numpy as jnp
from jax import lax
from jax.experimental import pallas as pl
from jax.experimental.pallas import tpu as pltpu
from jax.experimental.pallas import tpu_sc as plsc

VOCAB = 100000
D = 128
B = 16384
NC = 2
NS = 16
NW = NC * NS
BW = B // NW
C = 256
NCHUNK = BW // C


def _lane_permute(v, idx):
    dnums = lax.GatherDimensionNumbers(
        offset_dims=(), collapsed_slice_dims=(0,), start_index_map=(0,))
    return lax.gather(v, idx[:, None], dimension_numbers=dnums,
                      slice_sizes=(1,),
                      mode=lax.GatherScatterMode.PROMISE_IN_BOUNDS)


def _lane_sum(v, lane):
    for s in (1, 2, 4, 8):
        v = v + _lane_permute(v, lane ^ s)
    return v


def _glove_body(i_hbm, j_hbm, wi_hbm, wj_hbm, bi_hbm, bj_hbm, out_hbm,
                idx_i, idx_j, rows_i, rows_j, bi_v, bj_v, out_v,
                sem_rows, sem_bias):
    wid = lax.axis_index("s") * NC + lax.axis_index("c")
    base = wid * BW

    pltpu.sync_copy(i_hbm.at[pl.ds(base, BW)], idx_i)
    pltpu.sync_copy(j_hbm.at[pl.ds(base, BW)], idx_j)

    cb_i = pltpu.async_copy(bi_hbm.at[idx_i], bi_v, sem_bias)
    cb_j = pltpu.async_copy(bj_hbm.at[idx_j], bj_v, sem_bias)

    for c in range(NCHUNK):
        ci = pltpu.async_copy(wi_hbm.at[idx_i.at[pl.ds(c * C, C)]], rows_i,
                              sem_rows)
        cj = pltpu.async_copy(wj_hbm.at[idx_j.at[pl.ds(c * C, C)]], rows_j,
                              sem_rows)
        ci.wait()
        cj.wait()

        lane = lax.iota(jnp.int32, 16)

        def group_body(g, _, c=c):
            res = jnp.zeros((16,), jnp.float32)
            for l in range(16):
                r = g * 16 + l
                acc = rows_i[r, pl.ds(0, 16)] * rows_j[r, pl.ds(0, 16)]
                for k in range(1, 8):
                    acc = acc + (rows_i[r, pl.ds(16 * k, 16)]
                                 * rows_j[r, pl.ds(16 * k, 16)])
                res = jnp.where(lane == l, _lane_sum(acc, lane), res)
            out_v[pl.ds(c * C + g * 16, 16)] = res
            return 0

        lax.fori_loop(0, C // 16, group_body, 0)

    cb_i.wait()
    cb_j.wait()

    def bias_body(g, _):
        sl = pl.ds(g * 16, 16)
        out_v[sl] = out_v[sl] + bi_v[sl] + bj_v[sl]
        return 0

    lax.fori_loop(0, BW // 16, bias_body, 0, unroll=4)

    pltpu.sync_copy(out_v, out_hbm.at[pl.ds(base, BW)])


@jax.jit
def _glove(i, j, wi, wj, bi, bj):
    mesh = plsc.VectorSubcoreMesh(core_axis_name="c", subcore_axis_name="s",
                                  num_cores=NC, num_subcores=NS)
    run = pl.kernel(
        _glove_body,
        out_type=jax.ShapeDtypeStruct((B,), jnp.float32),
        mesh=mesh,
        scratch_types=[
            pltpu.VMEM((BW,), jnp.int32),
            pltpu.VMEM((BW,), jnp.int32),
            pltpu.VMEM((C, D), jnp.float32),
            pltpu.VMEM((C, D), jnp.float32),
            pltpu.VMEM((BW,), jnp.float32),
            pltpu.VMEM((BW,), jnp.float32),
            pltpu.VMEM((BW,), jnp.float32),
            pltpu.SemaphoreType.DMA,
            pltpu.SemaphoreType.DMA,
        ],
    )
    return run(i, j, wi, wj, bi, bj)


def kernel(i, j, wi, wj, bi, bj):
    return _glove(i, j, wi, wj, bi.reshape(VOCAB), bj.reshape(VOCAB))


# speedup vs baseline: 1.1914x; 1.0256x over previous
import functools

import jax
import jax.numpy as jnp
from jax import lax
from jax.experimental import pallas as pl
from jax.experimental.pallas import tpu as pltpu
from jax.experimental.pallas import tpu_sc as plsc

VOCAB = 100000
D = 128
B = 16384
NC = 2
NS = 16
NW = NC * NS
BW = B // NW
C = 128
NCHUNK = BW // C


def _lane_permute(v, idx):
    dnums = lax.GatherDimensionNumbers(
        offset_dims=(), collapsed_slice_dims=(0,), start_index_map=(0,))
    return lax.gather(v, idx[:, None], dimension_numbers=dnums,
                      slice_sizes=(1,),
                      mode=lax.GatherScatterMode.PROMISE_IN_BOUNDS)


def _lane_sum(v, lane):
    for s in (1, 2, 4, 8):
        v = v + _lane_permute(v, lane ^ s)
    return v


def _glove_body(i_hbm, j_hbm, wi_hbm, wj_hbm, bi_hbm, bj_hbm, out_hbm,
                idx_i, idx_j, ri0, ri1, rj0, rj1, bi_v, bj_v, out_v,
                sem0, sem1, sem_bias):
    wid = lax.axis_index("s") * NC + lax.axis_index("c")
    base = wid * BW

    pltpu.sync_copy(i_hbm.at[pl.ds(base, BW)], idx_i)
    pltpu.sync_copy(j_hbm.at[pl.ds(base, BW)], idx_j)

    cb_i = pltpu.async_copy(bi_hbm.at[idx_i], bi_v, sem_bias)
    cb_j = pltpu.async_copy(bj_hbm.at[idx_j], bj_v, sem_bias)

    ri = (ri0, ri1)
    rj = (rj0, rj1)
    sems = (sem0, sem1)
    lane = lax.iota(jnp.int32, 16)

    def fire(c):
        b = c % 2
        return (
            pltpu.async_copy(wi_hbm.at[idx_i.at[pl.ds(c * C, C)]], ri[b],
                             sems[b]),
            pltpu.async_copy(wj_hbm.at[idx_j.at[pl.ds(c * C, C)]], rj[b],
                             sems[b]),
        )

    inflight = fire(0)
    for c in range(NCHUNK):
        nxt = fire(c + 1) if c + 1 < NCHUNK else None
        ci, cj = inflight
        ci.wait()
        cj.wait()
        rows_i = ri[c % 2]
        rows_j = rj[c % 2]

        def group_body(g, _, c=c, rows_i=rows_i, rows_j=rows_j):
            res = jnp.zeros((16,), jnp.float32)
            for l in range(16):
                r = g * 16 + l
                acc = rows_i[r, pl.ds(0, 16)] * rows_j[r, pl.ds(0, 16)]
                for k in range(1, 8):
                    acc = acc + (rows_i[r, pl.ds(16 * k, 16)]
                                 * rows_j[r, pl.ds(16 * k, 16)])
                res = jnp.where(lane == l, _lane_sum(acc, lane), res)
            out_v[pl.ds(c * C + g * 16, 16)] = res
            return 0

        lax.fori_loop(0, C // 16, group_body, 0)
        inflight = nxt

    cb_i.wait()
    cb_j.wait()

    def bias_body(g, _):
        sl = pl.ds(g * 16, 16)
        out_v[sl] = out_v[sl] + bi_v[sl] + bj_v[sl]
        return 0

    lax.fori_loop(0, BW // 16, bias_body, 0, unroll=4)

    pltpu.sync_copy(out_v, out_hbm.at[pl.ds(base, BW)])


@jax.jit
def _glove(i, j, wi, wj, bi, bj):
    mesh = plsc.VectorSubcoreMesh(core_axis_name="c", subcore_axis_name="s",
                                  num_cores=NC, num_subcores=NS)
    run = pl.kernel(
        _glove_body,
        out_type=jax.ShapeDtypeStruct((B,), jnp.float32),
        mesh=mesh,
        scratch_types=[
            pltpu.VMEM((BW,), jnp.int32),
            pltpu.VMEM((BW,), jnp.int32),
            pltpu.VMEM((C, D), jnp.float32),
            pltpu.VMEM((C, D), jnp.float32),
            pltpu.VMEM((C, D), jnp.float32),
            pltpu.VMEM((C, D), jnp.float32),
            pltpu.VMEM((BW,), jnp.float32),
            pltpu.VMEM((BW,), jnp.float32),
            pltpu.VMEM((BW,), jnp.float32),
            pltpu.SemaphoreType.DMA,
            pltpu.SemaphoreType.DMA,
            pltpu.SemaphoreType.DMA,
        ],
    )
    return run(i, j, wi, wj, bi, bj)


def kernel(i, j, wi, wj, bi, bj):
    return _glove(i, j, wi, wj, bi.reshape(VOCAB), bj.reshape(VOCAB))


# speedup vs baseline: 1.2777x; 1.0725x over previous
import functools

import jax
import jax.numpy as jnp
from jax import lax
from jax.experimental import pallas as pl
from jax.experimental.pallas import tpu as pltpu
from jax.experimental.pallas import tpu_sc as plsc

VOCAB = 100000
D = 128
B = 16384
NC = 2
NS = 16
NW = NC * NS
BW = B // NW
C = 128
NCHUNK = BW // C


def _lane_permute(v, idx):
    dnums = lax.GatherDimensionNumbers(
        offset_dims=(), collapsed_slice_dims=(0,), start_index_map=(0,))
    return lax.gather(v, idx[:, None], dimension_numbers=dnums,
                      slice_sizes=(1,),
                      mode=lax.GatherScatterMode.PROMISE_IN_BOUNDS)


def _lane_sum(v, lane):
    for s in (1, 2, 4, 8):
        v = v + _lane_permute(v, lane ^ s)
    return v


def _glove_body(i_hbm, j_hbm, wi_hbm, wj_hbm, bi_hbm, bj_hbm, out_hbm,
                idx_i, idx_j, ri0, ri1, rj0, rj1, bi_v, bj_v, out_v,
                sem0, sem1, sem_bias):
    wid = lax.axis_index("s") * NC + lax.axis_index("c")
    base = wid * BW

    pltpu.sync_copy(i_hbm.at[pl.ds(base, BW)], idx_i)
    pltpu.sync_copy(j_hbm.at[pl.ds(base, BW)], idx_j)

    cb_i = pltpu.async_copy(bi_hbm.at[idx_i], bi_v, sem_bias)
    cb_j = pltpu.async_copy(bj_hbm.at[idx_j], bj_v, sem_bias)

    ri = (ri0, ri1)
    rj = (rj0, rj1)
    sems = (sem0, sem1)

    def fire(c):
        b = c % 2
        return (
            pltpu.async_copy(wi_hbm.at[idx_i.at[pl.ds(c * C, C)]], ri[b],
                             sems[b]),
            pltpu.async_copy(wj_hbm.at[idx_j.at[pl.ds(c * C, C)]], rj[b],
                             sems[b]),
        )

    inflight = fire(0)

    cb_i.wait()
    cb_j.wait()

    def bias_body(g, _):
        sl = pl.ds(g * 16, 16)
        out_v[sl] = bi_v[sl] + bj_v[sl]
        return 0

    lax.fori_loop(0, BW // 16, bias_body, 0)

    for c in range(NCHUNK):
        nxt = fire(c + 1) if c + 1 < NCHUNK else None
        ci, cj = inflight
        ci.wait()
        cj.wait()
        rows_i = ri[c % 2]
        rows_j = rj[c % 2]

        def row_body(r, _, c=c, rows_i=rows_i, rows_j=rows_j):
            acc0 = rows_i[r, pl.ds(0, 16)] * rows_j[r, pl.ds(0, 16)]
            acc1 = rows_i[r, pl.ds(16, 16)] * rows_j[r, pl.ds(16, 16)]
            for k in range(2, 8, 2):
                acc0 = acc0 + (rows_i[r, pl.ds(16 * k, 16)]
                               * rows_j[r, pl.ds(16 * k, 16)])
                acc1 = acc1 + (rows_i[r, pl.ds(16 * (k + 1), 16)]
                               * rows_j[r, pl.ds(16 * (k + 1), 16)])
            ridx = jnp.broadcast_to(c * C + r, (16,)).astype(jnp.int32)
            plsc.addupdate_scatter(out_v, [ridx], acc0 + acc1)
            return 0

        lax.fori_loop(0, C, row_body, 0)
        inflight = nxt

    pltpu.sync_copy(out_v, out_hbm.at[pl.ds(base, BW)])


@jax.jit
def _glove(i, j, wi, wj, bi, bj):
    mesh = plsc.VectorSubcoreMesh(core_axis_name="c", subcore_axis_name="s",
                                  num_cores=NC, num_subcores=NS)
    run = pl.kernel(
        _glove_body,
        out_type=jax.ShapeDtypeStruct((B,), jnp.float32),
        mesh=mesh,
        compiler_params=pltpu.CompilerParams(needs_layout_passes=False),
        scratch_types=[
            pltpu.VMEM((BW,), jnp.int32),
            pltpu.VMEM((BW,), jnp.int32),
            pltpu.VMEM((C, D), jnp.float32),
            pltpu.VMEM((C, D), jnp.float32),
            pltpu.VMEM((C, D), jnp.float32),
            pltpu.VMEM((C, D), jnp.float32),
            pltpu.VMEM((BW,), jnp.float32),
            pltpu.VMEM((BW,), jnp.float32),
            pltpu.VMEM((BW,), jnp.float32),
            pltpu.SemaphoreType.DMA,
            pltpu.SemaphoreType.DMA,
            pltpu.SemaphoreType.DMA,
        ],
    )
    return run(i, j, wi, wj, bi, bj)


def kernel(i, j, wi, wj, bi, bj):
    return _glove(i, j, wi, wj, bi.reshape(VOCAB), bj.reshape(VOCAB))
